# baseline (speedup 1.0000x reference)
import functools

import jax
import jax.numpy as jnp
import numpy as np
from jax.experimental import pallas as pl
from jax.experimental.pallas import tpu as pltpu

_BINS = 10
_DELTA = 0.1
_SIGMA = 150.0
_LOG2E = 1.4426950408889634
_K2 = _SIGMA * _LOG2E
_SHIFT = 108.0
_EDGE_C = [float(2.0 ** (_K2 * _DELTA * i - _SHIFT)) for i in range(_BINS + 1)]

_ROWS = 48
_H = 512
_W = 512
_NCORES = 2
_RBLK = _ROWS // _NCORES
_HBLK = 32
_KSTEPS = _H // _HBLK
_ACC_LANES = 16

_RSEL = np.zeros((_W * (_BINS + 1), _ACC_LANES), dtype=np.float32)
for _i in range(_BINS + 1):
    _RSEL[_W * _i:_W * (_i + 1), _i] = 1.0


def _edge_sums_body(acc_vmem, rsel_vmem, x_blk, y_blk):
    xb = x_blk[...]
    yb = y_blk[...]
    px = jnp.exp2(_K2 * xb - _SHIFT).astype(jnp.bfloat16)
    py = jnp.exp2(_K2 * yb - _SHIFT).astype(jnp.bfloat16)
    cl3 = jnp.bfloat16(2.0 ** (-_K2 * _DELTA))
    ch3 = jnp.bfloat16(2.0 ** (+_K2 * _DELTA))
    one3 = jnp.bfloat16(1.0)
    ucap3 = jnp.bfloat16(2.0 ** 39)
    cl2 = jnp.bfloat16(2.0 ** (-_K2 * _DELTA / 2))
    ch2 = jnp.bfloat16(2.0 ** (+_K2 * _DELTA / 2))

    def _triple_sigs(p, k):
        inv_cgeo = jnp.bfloat16(2.0 ** (_SHIFT - _K2 * _DELTA * (3 * k + 1)))
        u = jnp.minimum(p * inv_cgeo, ucap3)
        f1 = u + cl3
        f2 = u + one3
        f3 = u + ch3
        g23 = f2 * f3
        g13 = f1 * f3
        g12 = f1 * f2
        den = f1 * g23
        return (u * g23) / den, (u * g13) / den, (u * g12) / den

    def _last_pair_sigs(p):
        inv_cgeo = jnp.bfloat16(2.0 ** (_SHIFT - _K2 * _DELTA * 9.5))
        u = p * inv_cgeo
        al = u + cl2
        ah = u + ch2
        den = al * ah
        return (u * ah) / den, (u * al) / den

    sums = None
    for k in range(3):
        sx0, sx1, sx2 = _triple_sigs(px, k)
        sy0, sy1, sy2 = _triple_sigs(py, k)
        dk = jnp.concatenate([sx0 - sy0, sx1 - sy1, sx2 - sy2], axis=2)
        dk_view = dk.reshape(_RBLK * _HBLK, 3 * _W)
        r = rsel_vmem[3 * _W * k:3 * _W * (k + 1), :]
        m = jnp.dot(dk_view, r, preferred_element_type=jnp.float32)
        sums = m if sums is None else sums + m
    sx9, sx10 = _last_pair_sigs(px)
    sy9, sy10 = _last_pair_sigs(py)
    dk = jnp.concatenate([sx9 - sy9, sx10 - sy10], axis=2)
    dk_view = dk.reshape(_RBLK * _HBLK, 2 * _W)
    r = rsel_vmem[9 * _W:11 * _W, :]
    sums = sums + jnp.dot(dk_view, r, preferred_element_type=jnp.float32)
    part = jnp.sum(sums.reshape(_RBLK, _HBLK, _ACC_LANES), axis=1)
    acc_vmem[...] += part


def _core_worker(x_ref, y_ref, rsel_ref, acc_ref, acc_vmem, rsel_vmem, sem):
    core = jax.lax.axis_index("core")
    cp_in = pltpu.make_async_copy(rsel_ref, rsel_vmem, sem)
    cp_in.start()
    cp_in.wait()
    acc_vmem[...] = jnp.zeros((_RBLK, _ACC_LANES), jnp.float32)
    pltpu.emit_pipeline(
        functools.partial(_edge_sums_body, acc_vmem, rsel_vmem),
        grid=(_NCORES, _KSTEPS),
        in_specs=[
            pl.BlockSpec((_RBLK, _HBLK, _W), lambda g, k: (g, k, 0)),
            pl.BlockSpec((_RBLK, _HBLK, _W), lambda g, k: (g, k, 0)),
        ],
        core_axis_name="core",
        dimension_semantics=(pltpu.PARALLEL, pltpu.ARBITRARY),
    )(x_ref, y_ref)
    cp = pltpu.make_async_copy(
        acc_vmem, acc_ref.at[pl.ds(core * _RBLK, _RBLK)], sem)
    cp.start()
    cp.wait()


def _loss_kernel(acc_ref, out_ref):
    u = acc_ref[...]
    d = u[:, 0:_BINS] - u[:, 1:_BINS + 1]
    total = jnp.sum(jnp.abs(d), axis=(0, 1), keepdims=True)
    out_ref[...] = total * (0.0001 / (_BINS * 16))


def kernel(x, y):
    xr = x.reshape(_ROWS, _H, _W)
    yr = y.reshape(_ROWS, _H, _W)

    mesh = pltpu.create_tensorcore_mesh("core", num_cores=_NCORES)

    def run(refs):
        x_ref, y_ref, rsel_ref, acc_ref = refs

        @pl.core_map(mesh)
        def _():
            pl.run_scoped(
                functools.partial(_core_worker, x_ref, y_ref, rsel_ref, acc_ref),
                pltpu.VMEM((_RBLK, _ACC_LANES), jnp.float32),
                pltpu.VMEM((_W * (_BINS + 1), _ACC_LANES), jnp.bfloat16),
                pltpu.SemaphoreType.DMA,
            )

    _, _, _, acc = pl.run_state(run)(
        (xr, yr, jnp.asarray(_RSEL, dtype=jnp.bfloat16),
         jnp.zeros((_ROWS, _ACC_LANES), jnp.float32)))

    out = pl.pallas_call(
        _loss_kernel,
        out_shape=jax.ShapeDtypeStruct((1, 1), jnp.float32),
    )(acc)
    return out[0, 0]


# speedup vs baseline: 1.0602x; 1.0602x over previous
import functools

import jax
import jax.numpy as jnp
import numpy as np
from jax.experimental import pallas as pl
from jax.experimental.pallas import tpu as pltpu

_BINS = 10
_DELTA = 0.1
_SIGMA = 150.0
_LOG2E = 1.4426950408889634
_K2 = _SIGMA * _LOG2E
_SHIFT = 108.0
_EDGE_C = [float(2.0 ** (_K2 * _DELTA * i - _SHIFT)) for i in range(_BINS + 1)]

_ROWS = 48
_H = 512
_W = 512
_NCORES = 2
_RBLK = _ROWS // _NCORES
_HBLK = 64
_KSTEPS = _H // _HBLK
_ACC_LANES = 16

_RSEL = np.zeros((_W * (_BINS + 1), _ACC_LANES), dtype=np.float32)
for _i in range(_BINS + 1):
    _RSEL[_W * _i:_W * (_i + 1), _i] = 1.0


def _edge_sums_body(acc_vmem, rsel_vmem, x_blk, y_blk):
    xb = x_blk[...]
    yb = y_blk[...]
    px = jnp.exp2(_K2 * xb - _SHIFT).astype(jnp.bfloat16)
    py = jnp.exp2(_K2 * yb - _SHIFT).astype(jnp.bfloat16)
    cl = jnp.bfloat16(2.0 ** (-_K2 * _DELTA / 2))
    ch = jnp.bfloat16(2.0 ** (+_K2 * _DELTA / 2))
    ucap = jnp.bfloat16(2.0 ** 40)

    def _pair_sigs(p, k):
        inv_cgeo = jnp.bfloat16(2.0 ** (_SHIFT - _K2 * _DELTA * (2 * k + 0.5)))
        u = p * inv_cgeo
        if k < 4:
            u = jnp.minimum(u, ucap)
        al = u + cl
        ah = u + ch
        den = al * ah
        return (u * ah) / den, (u * al) / den

    sums = None
    for k in range(5):
        sxl, sxh = _pair_sigs(px, k)
        syl, syh = _pair_sigs(py, k)
        dk = jnp.concatenate([sxl - syl, sxh - syh], axis=2)
        dk_view = dk.reshape(_RBLK * _HBLK, 2 * _W)
        r = rsel_vmem[2 * _W * k:2 * _W * (k + 1), :]
        m = jnp.dot(dk_view, r, preferred_element_type=jnp.float32)
        sums = m if sums is None else sums + m
    c10 = jnp.bfloat16(_EDGE_C[_BINS])
    d10 = (px / (px + c10)) - (py / (py + c10))
    d10_view = d10.reshape(_RBLK * _HBLK, _W)
    r10 = rsel_vmem[2 * _W * 5:2 * _W * 5 + _W, :]
    sums = sums + jnp.dot(d10_view, r10, preferred_element_type=jnp.float32)
    part = jnp.sum(sums.reshape(_RBLK, _HBLK, _ACC_LANES), axis=1)
    acc_vmem[...] += part


def _core_worker(x_ref, y_ref, rsel_ref, acc_ref, acc_vmem, rsel_vmem, sem):
    core = jax.lax.axis_index("core")
    cp_in = pltpu.make_async_copy(rsel_ref, rsel_vmem, sem)
    cp_in.start()
    cp_in.wait()
    acc_vmem[...] = jnp.zeros((_RBLK, _ACC_LANES), jnp.float32)
    pltpu.emit_pipeline(
        functools.partial(_edge_sums_body, acc_vmem, rsel_vmem),
        grid=(_NCORES, _KSTEPS),
        in_specs=[
            pl.BlockSpec((_RBLK, _HBLK, _W), lambda g, k: (g, k, 0)),
            pl.BlockSpec((_RBLK, _HBLK, _W), lambda g, k: (g, k, 0)),
        ],
        core_axis_name="core",
        dimension_semantics=(pltpu.PARALLEL, pltpu.ARBITRARY),
    )(x_ref, y_ref)
    cp = pltpu.make_async_copy(
        acc_vmem, acc_ref.at[pl.ds(core * _RBLK, _RBLK)], sem)
    cp.start()
    cp.wait()


def _loss_kernel(acc_ref, out_ref):
    u = acc_ref[...]
    d = u[:, 0:_BINS] - u[:, 1:_BINS + 1]
    total = jnp.sum(jnp.abs(d), axis=(0, 1), keepdims=True)
    out_ref[...] = total * (0.0001 / (_BINS * 16))


def kernel(x, y):
    xr = x.reshape(_ROWS, _H, _W)
    yr = y.reshape(_ROWS, _H, _W)

    mesh = pltpu.create_tensorcore_mesh("core", num_cores=_NCORES)

    def run(refs):
        x_ref, y_ref, rsel_ref, acc_ref = refs

        @pl.core_map(mesh)
        def _():
            pl.run_scoped(
                functools.partial(_core_worker, x_ref, y_ref, rsel_ref, acc_ref),
                pltpu.VMEM((_RBLK, _ACC_LANES), jnp.float32),
                pltpu.VMEM((_W * (_BINS + 1), _ACC_LANES), jnp.bfloat16),
                pltpu.SemaphoreType.DMA,
            )

    _, _, _, acc = pl.run_state(run)(
        (xr, yr, jnp.asarray(_RSEL, dtype=jnp.bfloat16),
         jnp.zeros((_ROWS, _ACC_LANES), jnp.float32)))

    out = pl.pallas_call(
        _loss_kernel,
        out_shape=jax.ShapeDtypeStruct((1, 1), jnp.float32),
    )(acc)
    return out[0, 0]


# speedup vs baseline: 1.0983x; 1.0360x over previous
import functools

import jax
import jax.numpy as jnp
import numpy as np
from jax.experimental import pallas as pl
from jax.experimental.pallas import tpu as pltpu

_BINS = 10
_DELTA = 0.1
_SIGMA = 150.0
_LOG2E = 1.4426950408889634
_K2 = _SIGMA * _LOG2E
_SHIFT = 108.0
_EDGE_C = [float(2.0 ** (_K2 * _DELTA * i - _SHIFT)) for i in range(_BINS + 1)]

_ROWS = 48
_H = 512
_W = 512
_NCORES = 2
_RBLK = _ROWS // _NCORES
_HBLK = 32
_KSTEPS = _H // _HBLK
_ACC_LANES = 16

_RSEL = np.zeros((_W * (_BINS + 1), _ACC_LANES), dtype=np.float32)
for _i in range(_BINS + 1):
    _RSEL[_W * _i:_W * (_i + 1), _i] = 1.0


def _edge_sums_body(acc_vmem, rsel_vmem, x_blk, y_blk):
    xb = x_blk[...]
    yb = y_blk[...]
    px = jnp.exp2(_K2 * xb - _SHIFT).astype(jnp.bfloat16)
    py = jnp.exp2(_K2 * yb - _SHIFT).astype(jnp.bfloat16)
    cl = jnp.bfloat16(2.0 ** (-_K2 * _DELTA / 2))
    ch = jnp.bfloat16(2.0 ** (+_K2 * _DELTA / 2))
    ucap = jnp.bfloat16(2.0 ** 40)

    def _pair_sigs(p, k):
        inv_cgeo = jnp.bfloat16(2.0 ** (_SHIFT - _K2 * _DELTA * (2 * k + 0.5)))
        u = p * inv_cgeo
        if k < 4:
            u = jnp.minimum(u, ucap)
        al = u + cl
        ah = u + ch
        den = al * ah
        return (u * ah) / den, (u * al) / den

    sums = None
    for k in range(5):
        sxl, sxh = _pair_sigs(px, k)
        syl, syh = _pair_sigs(py, k)
        dk = jnp.concatenate([sxl - syl, sxh - syh], axis=2)
        dk_view = dk.reshape(_RBLK * _HBLK, 2 * _W)
        r = rsel_vmem[2 * _W * k:2 * _W * (k + 1), :]
        m = jnp.dot(dk_view, r, preferred_element_type=jnp.float32)
        sums = m if sums is None else sums + m
    c10 = jnp.bfloat16(_EDGE_C[_BINS])
    d10 = (px / (px + c10)) - (py / (py + c10))
    d10_view = d10.reshape(_RBLK * _HBLK, _W)
    r10 = rsel_vmem[2 * _W * 5:2 * _W * 5 + _W, :]
    sums = sums + jnp.dot(d10_view, r10, preferred_element_type=jnp.float32)
    part = jnp.sum(sums.reshape(_RBLK, _HBLK, _ACC_LANES), axis=1)
    acc_vmem[...] += part


def _core_worker(x_ref, y_ref, rsel_ref, acc_ref, acc_vmem, rsel_vmem, sem):
    core = jax.lax.axis_index("core")
    cp_in = pltpu.make_async_copy(rsel_ref, rsel_vmem, sem)
    cp_in.start()
    cp_in.wait()
    acc_vmem[...] = jnp.zeros((_RBLK, _ACC_LANES), jnp.float32)
    pltpu.emit_pipeline(
        functools.partial(_edge_sums_body, acc_vmem, rsel_vmem),
        grid=(_NCORES, _KSTEPS),
        in_specs=[
            pl.BlockSpec((_RBLK, _HBLK, _W), lambda g, k: (g, k, 0)),
            pl.BlockSpec((_RBLK, _HBLK, _W), lambda g, k: (g, k, 0)),
        ],
        core_axis_name="core",
        dimension_semantics=(pltpu.PARALLEL, pltpu.ARBITRARY),
    )(x_ref, y_ref)
    cp = pltpu.make_async_copy(
        acc_vmem, acc_ref.at[pl.ds(core * _RBLK, _RBLK)], sem)
    cp.start()
    cp.wait()


def _loss_kernel(acc_ref, out_ref):
    u = acc_ref[...]
    d = u[:, 0:_BINS] - u[:, 1:_BINS + 1]
    total = jnp.sum(jnp.abs(d), axis=(0, 1), keepdims=True)
    out_ref[...] = total * (0.0001 / (_BINS * 16))


def kernel(x, y):
    xr = x.reshape(_ROWS, _H, _W)
    yr = y.reshape(_ROWS, _H, _W)

    mesh = pltpu.create_tensorcore_mesh("core", num_cores=_NCORES)

    def run(refs):
        x_ref, y_ref, rsel_ref, acc_ref = refs

        @pl.core_map(mesh)
        def _():
            pl.run_scoped(
                functools.partial(_core_worker, x_ref, y_ref, rsel_ref, acc_ref),
                pltpu.VMEM((_RBLK, _ACC_LANES), jnp.float32),
                pltpu.VMEM((_W * (_BINS + 1), _ACC_LANES), jnp.bfloat16),
                pltpu.SemaphoreType.DMA,
            )

    _, _, _, acc = pl.run_state(run)(
        (xr, yr, jnp.asarray(_RSEL, dtype=jnp.bfloat16),
         jnp.zeros((_ROWS, _ACC_LANES), jnp.float32)))

    out = pl.pallas_call(
        _loss_kernel,
        out_shape=jax.ShapeDtypeStruct((1, 1), jnp.float32),
    )(acc)
    return out[0, 0]
